# baseline (speedup 1.0000x reference)
import jax
import jax.numpy as jnp
from jax.experimental import pallas as pl
from jax.experimental.pallas import tpu as pltpu

_TJ = 1024


def _gcn_kernel(x_ref, a_ref, w1_ref, b1_ref, w2_ref, b2_ref,
                out_ref, y_s, h_s, c_s):
    nj = 4096 // _TJ
    j = pl.program_id(0) % nj
    n = a_ref.shape[2]

    @pl.when(j == 0)
    def _init():
        y_s[...] = jnp.dot(x_ref[0], w1_ref[...],
                           preferred_element_type=jnp.float32)
        c_s[...] = jnp.zeros_like(c_s)

    c_s[...] += jnp.sum(a_ref[0], axis=0, keepdims=True)
    h = (jnp.dot(a_ref[0], y_s[...], preferred_element_type=jnp.float32)
         + b1_ref[...])
    h = jnp.where(h >= 0, h, 0.01 * h)
    h_s[pl.ds(j * _TJ, _TJ), :] = h

    @pl.when(j == nj - 1)
    def _finish():
        ch = jnp.dot(c_s[...], h_s[...], preferred_element_type=jnp.float32)
        out = jnp.dot(ch, w2_ref[...], preferred_element_type=jnp.float32)
        out_ref[0] = out * (1.0 / n) + b2_ref[...]


def kernel(x, graph_batch, W1, b1, W2, b2):
    B, N, F = x.shape
    Hc = W1.shape[1]
    O = W2.shape[1]
    b1r = b1.reshape(1, Hc)
    b2r = b2.reshape(1, O)
    out = pl.pallas_call(
        _gcn_kernel,
        grid=(B * (N // _TJ),),
        in_specs=[
            pl.BlockSpec((1, N, F), lambda t: (t // (N // _TJ), 0, 0)),
            pl.BlockSpec((1, _TJ, N),
                         lambda t: (t // (N // _TJ), t % (N // _TJ), 0)),
            pl.BlockSpec((F, Hc), lambda t: (0, 0)),
            pl.BlockSpec((1, Hc), lambda t: (0, 0)),
            pl.BlockSpec((Hc, O), lambda t: (0, 0)),
            pl.BlockSpec((1, O), lambda t: (0, 0)),
        ],
        out_specs=pl.BlockSpec((1, 1, O), lambda t: (t // (N // _TJ), 0, 0)),
        out_shape=jax.ShapeDtypeStruct((B, 1, O), jnp.float32),
        scratch_shapes=[
            pltpu.VMEM((N, Hc), jnp.float32),
            pltpu.VMEM((N, Hc), jnp.float32),
            pltpu.VMEM((1, N), jnp.float32),
        ],
    )(x, graph_batch, W1, b1r, W2, b2r)
    return out.reshape(B, O)


# speedup vs baseline: 1.0012x; 1.0012x over previous
import jax
import jax.numpy as jnp
from jax.experimental import pallas as pl
from jax.experimental.pallas import tpu as pltpu

_TJ = 1024


def _gcn_kernel(x_ref, a_ref, w1_ref, b1_ref, w2_ref, b2_ref,
                out_ref, y_s, h_s, c_s):
    j = pl.program_id(1)
    nj = pl.num_programs(1)
    n = a_ref.shape[2]

    @pl.when(j == 0)
    def _init():
        y_s[...] = jnp.dot(x_ref[0], w1_ref[...],
                           preferred_element_type=jnp.float32)
        c_s[...] = jnp.zeros_like(c_s)

    c_s[...] += jnp.sum(a_ref[0], axis=0, keepdims=True)
    h = (jnp.dot(a_ref[0], y_s[...], preferred_element_type=jnp.float32)
         + b1_ref[...])
    h = jnp.where(h >= 0, h, 0.01 * h)
    h_s[pl.ds(j * _TJ, _TJ), :] = h

    @pl.when(j == nj - 1)
    def _finish():
        ch = jnp.dot(c_s[...], h_s[...], preferred_element_type=jnp.float32)
        out = jnp.dot(ch, w2_ref[...], preferred_element_type=jnp.float32)
        out_ref[0] = out * (1.0 / n) + b2_ref[...]


def kernel(x, graph_batch, W1, b1, W2, b2):
    B, N, F = x.shape
    Hc = W1.shape[1]
    O = W2.shape[1]
    b1r = b1.reshape(1, Hc)
    b2r = b2.reshape(1, O)
    out = pl.pallas_call(
        _gcn_kernel,
        grid=(B, N // _TJ),
        in_specs=[
            pl.BlockSpec((1, N, F), lambda b, j: (b, 0, 0)),
            pl.BlockSpec((1, _TJ, N), lambda b, j: (b, j, 0)),
            pl.BlockSpec((F, Hc), lambda b, j: (0, 0)),
            pl.BlockSpec((1, Hc), lambda b, j: (0, 0)),
            pl.BlockSpec((Hc, O), lambda b, j: (0, 0)),
            pl.BlockSpec((1, O), lambda b, j: (0, 0)),
        ],
        out_specs=pl.BlockSpec((1, 1, O), lambda b, j: (b, 0, 0)),
        out_shape=jax.ShapeDtypeStruct((B, 1, O), jnp.float32),
        scratch_shapes=[
            pltpu.VMEM((N, Hc), jnp.float32),
            pltpu.VMEM((N, Hc), jnp.float32),
            pltpu.VMEM((1, N), jnp.float32),
        ],
    )(x, graph_batch, W1, b1r, W2, b2r)
    return out.reshape(B, O)


# speedup vs baseline: 1.0104x; 1.0092x over previous
import jax
import jax.numpy as jnp
from jax.experimental import pallas as pl
from jax.experimental.pallas import tpu as pltpu

_TJ = 1024


def _gcn_kernel(x_ref, a_ref, w1_ref, b1_ref, w2_ref, b2_ref,
                out_ref, y_s, h_s, c_s):
    j = pl.program_id(1)
    nj = pl.num_programs(1)
    n = a_ref.shape[2]

    @pl.when(j == 0)
    def _init():
        y_s[...] = jnp.dot(x_ref[pl.program_id(0)], w1_ref[...],
                           preferred_element_type=jnp.float32)
        c_s[...] = jnp.zeros_like(c_s)

    c_s[...] += jnp.sum(a_ref[0], axis=0, keepdims=True)
    h = (jnp.dot(a_ref[0], y_s[...], preferred_element_type=jnp.float32)
         + b1_ref[...])
    h = jnp.where(h >= 0, h, 0.01 * h)
    h_s[pl.ds(j * _TJ, _TJ), :] = h

    @pl.when(j == nj - 1)
    def _finish():
        ch = jnp.dot(c_s[...], h_s[...], preferred_element_type=jnp.float32)
        out = jnp.dot(ch, w2_ref[...], preferred_element_type=jnp.float32)
        out_ref[0] = out * (1.0 / n) + b2_ref[...]


def kernel(x, graph_batch, W1, b1, W2, b2):
    B, N, F = x.shape
    Hc = W1.shape[1]
    O = W2.shape[1]
    b1r = b1.reshape(1, Hc)
    b2r = b2.reshape(1, O)
    out = pl.pallas_call(
        _gcn_kernel,
        grid=(B, N // _TJ),
        in_specs=[
            pl.BlockSpec((B, N, F), lambda b, j: (0, 0, 0)),
            pl.BlockSpec((1, _TJ, N), lambda b, j: (b, j, 0)),
            pl.BlockSpec((F, Hc), lambda b, j: (0, 0)),
            pl.BlockSpec((1, Hc), lambda b, j: (0, 0)),
            pl.BlockSpec((Hc, O), lambda b, j: (0, 0)),
            pl.BlockSpec((1, O), lambda b, j: (0, 0)),
        ],
        out_specs=pl.BlockSpec((1, 1, O), lambda b, j: (b, 0, 0)),
        out_shape=jax.ShapeDtypeStruct((B, 1, O), jnp.float32),
        scratch_shapes=[
            pltpu.VMEM((N, Hc), jnp.float32),
            pltpu.VMEM((N, Hc), jnp.float32),
            pltpu.VMEM((1, N), jnp.float32),
        ],
    )(x, graph_batch, W1, b1r, W2, b2r)
    return out.reshape(B, O)
